# baseline (speedup 1.0000x reference)
import functools

import jax
import jax.numpy as jnp
from jax import lax
from jax.experimental import pallas as pl
from jax.experimental.pallas import tpu as pltpu
from jax.experimental.pallas import tpu_sc as plsc

N_NODES = 10000
N_EDGES = 320000
DIM = 128
BN_EPS = 1e-5

NC = 2
NS = 16
NW = NC * NS
E_PAD = 327680
CH = 128
NCHT = E_PAD // CH
SUP = 8
NSUP = NCHT // SUP
PASS_E = 8192
NPASS = E_PAD // PASS_E
SUP_PER_PASS = PASS_E // (SUP * CH)
N_ACC = 10240
RNG = N_ACC // NW
SL = PASS_E + CH

ROW_BLK = 2000
N_BLKS = N_NODES // ROW_BLK


def _mm_body(h_ref, w_ref, b_ref, o_ref):
    o_ref[...] = (
        jnp.dot(h_ref[...], w_ref[...], preferred_element_type=jnp.float32)
        + b_ref[...]
    )


def _project(h, W, b2):
    return pl.pallas_call(
        _mm_body,
        grid=(N_BLKS,),
        in_specs=[
            pl.BlockSpec((ROW_BLK, DIM), lambda i: (i, 0)),
            pl.BlockSpec((DIM, DIM), lambda i: (0, 0)),
            pl.BlockSpec((1, DIM), lambda i: (0, 0)),
        ],
        out_specs=pl.BlockSpec((ROW_BLK, DIM), lambda i: (i, 0)),
        out_shape=jax.ShapeDtypeStruct((N_NODES, DIM), jnp.float32),
    )(h, W, b2)


def _sc_aggregate(h2, edata):
    mesh = plsc.VectorSubcoreMesh(
        core_axis_name="c", subcore_axis_name="s", num_cores=NC, num_subcores=NS
    )

    @functools.partial(
        pl.kernel,
        out_type=jax.ShapeDtypeStruct((N_ACC, DIM), jnp.float32),
        mesh=mesh,
        scratch_types=[
            pltpu.VMEM((SUP * 3 * CH,), jnp.int32),
            pltpu.VMEM((SUP * 3 * CH,), jnp.int32),
            pltpu.VMEM((CH, DIM), jnp.float32),
            pltpu.VMEM((CH, DIM), jnp.float32),
            pltpu.VMEM((SL,), jnp.int32),
            pltpu.VMEM((SL,), jnp.int32),
            pltpu.VMEM((SL,), jnp.float32),
            pltpu.VMEM((RNG, DIM), jnp.float32),
            pltpu.SemaphoreType.DMA,
            pltpu.SemaphoreType.DMA,
            pltpu.SemaphoreType.DMA,
            pltpu.SemaphoreType.DMA,
        ],
        compiler_params=pltpu.CompilerParams(needs_layout_passes=False),
    )
    def body(h2_hbm, ed_hbm, out_hbm,
             eb0, eb1, rows0, rows1, slist, dlist, wlist, acc,
             sm0, sm1, sg0, sg1):
        c = lax.axis_index("c")
        s = lax.axis_index("s")
        g = c * NS + s
        gbase = g * RNG

        eb = (eb0, eb1)
        rows = (rows0, rows1)
        sm = (sm0, sm1)
        sg = (sg0, sg1)
        z16f = jnp.zeros((16,), jnp.float32)
        z16i = jnp.zeros((16,), jnp.int32)

        def zrow(r, _):
            for j in range(DIM // 16):
                acc[r, pl.ds(j * 16, 16)] = z16f
            return 0
        lax.fori_loop(0, RNG, zrow, 0)

        SUPW = SUP * 3 * CH

        def fire_meta(u, p):
            pltpu.async_copy(ed_hbm.at[pl.ds(u * SUPW, SUPW)], eb[p], sm[p])

        def wait_meta(u, p):
            pltpu.make_async_copy(ed_hbm.at[pl.ds(u * SUPW, SUPW)], eb[p],
                                  sm[p]).wait()

        def fire_g(q, p):
            pltpu.async_copy(h2_hbm.at[slist.at[pl.ds(q * CH, CH)]],
                             rows[p], sg[p])

        def wait_g(q, p):
            pltpu.make_async_copy(h2_hbm.at[slist.at[pl.ds(q * CH, CH)]],
                                  rows[p], sg[p]).wait()

        lo = gbase
        hi = gbase + RNG

        def scan_vreg(ebuf, cnt, i):
            base = (i >> 3) * (3 * CH) + (i & 7) * 16
            dv = ebuf[pl.ds(base + CH, 16)]
            m = jnp.logical_and(dv >= lo, dv < hi)
            sv = ebuf[pl.ds(base, 16)]
            wv = ebuf[pl.ds(base + 2 * CH, 16)]
            plsc.store_compressed(slist.at[pl.ds(cnt, 16)], sv, mask=m)
            plsc.store_compressed(dlist.at[pl.ds(cnt, 16)], dv, mask=m)
            plsc.store_compressed(wlist.at[pl.ds(cnt, 16)],
                                  plsc.bitcast(wv, jnp.float32), mask=m)
            return cnt + plsc.all_reduce_population_count(m)[0]

        def proc_chunk(q, rbuf):
            qoff = q * CH

            def grp(gi, _):
                goff = qoff + gi * 16
                dvec = dlist[pl.ds(goff, 16)]
                wvec = wlist[pl.ds(goff, 16)]
                for r in range(16):
                    local = dvec[r] - gbase
                    ws = lax.broadcast(wvec[r], (16,))
                    row = gi * 16 + r
                    for j in range(DIM // 16):
                        sl = pl.ds(j * 16, 16)
                        acc[local, sl] = acc[local, sl] + rbuf[row, sl] * ws
                return 0
            lax.fori_loop(0, CH // 16, grp, 0)

        fire_meta(0, 0)

        def pass_body(p, _):
            cnt = jnp.int32(0)
            for u in range(SUP_PER_PASS):
                usup = p * SUP_PER_PASS + u

                @pl.when(usup + 1 < NSUP)
                def _():
                    fire_meta(usup + 1, (u + 1) % 2)
                wait_meta(usup, u % 2)

                def sv_body(i, cc):
                    return scan_vreg(eb[u % 2], cc, i)
                cnt = lax.fori_loop(0, SUP * CH // 16, sv_body, cnt)

            for j in range(CH // 16):
                sl = pl.ds(cnt + j * 16, 16)
                slist[sl] = z16i
                dlist[sl] = jnp.full((16,), 1, jnp.int32) * gbase
                wlist[sl] = z16f

            nch = (cnt + (CH - 1)) >> 7

            @pl.when(nch > 0)
            def _():
                fire_g(0, 0)

            def pairb(t, _):
                q0 = 2 * t
                q1 = q0 + 1

                @pl.when(q1 < nch)
                def _():
                    fire_g(q1, 1)
                wait_g(q0, 0)
                proc_chunk(q0, rows[0])

                @pl.when(q1 < nch)
                def _():
                    @pl.when(q1 + 1 < nch)
                    def _():
                        fire_g(q1 + 1, 0)
                    wait_g(q1, 1)
                    proc_chunk(q1, rows[1])
                return 0
            lax.fori_loop(0, (nch + 1) >> 1, pairb, 0)
            return 0
        lax.fori_loop(0, NPASS, pass_body, 0)

        pltpu.sync_copy(acc, out_hbm.at[pl.ds(gbase, RNG)])

    return body(h2, edata)


def _comb_body(h2_ref, p0_ref, h3_ref, sum_ref, sq_ref):
    i = pl.program_id(0)
    x = h2_ref[...] + p0_ref[...]
    h3_ref[...] = x

    @pl.when(i == 0)
    def _():
        sum_ref[...] = jnp.zeros_like(sum_ref)
        sq_ref[...] = jnp.zeros_like(sq_ref)

    sum_ref[0:1, :] += jnp.sum(x, axis=0, keepdims=True)
    sq_ref[0:1, :] += jnp.sum(x * x, axis=0, keepdims=True)


def _combine(h2, p0):
    return pl.pallas_call(
        _comb_body,
        grid=(N_BLKS,),
        in_specs=[
            pl.BlockSpec((ROW_BLK, DIM), lambda i: (i, 0)),
            pl.BlockSpec((ROW_BLK, DIM), lambda i: (i, 0)),
        ],
        out_specs=[
            pl.BlockSpec((ROW_BLK, DIM), lambda i: (i, 0)),
            pl.BlockSpec((8, DIM), lambda i: (0, 0)),
            pl.BlockSpec((8, DIM), lambda i: (0, 0)),
        ],
        out_shape=[
            jax.ShapeDtypeStruct((N_NODES, DIM), jnp.float32),
            jax.ShapeDtypeStruct((8, DIM), jnp.float32),
            jax.ShapeDtypeStruct((8, DIM), jnp.float32),
        ],
    )(h2, p0)


def _bn_body(h3_ref, sum_ref, sq_ref, g_ref, be_ref, o_ref):
    n = jnp.float32(N_NODES)
    mean = sum_ref[0:1, :] / n
    var = sq_ref[0:1, :] / n - mean * mean
    inv = lax.rsqrt(var + BN_EPS)
    o_ref[...] = g_ref[...] * (h3_ref[...] - mean) * inv + be_ref[...]


def _bn_apply(h3, ssum, ssq, g2, be2):
    return pl.pallas_call(
        _bn_body,
        grid=(N_BLKS,),
        in_specs=[
            pl.BlockSpec((ROW_BLK, DIM), lambda i: (i, 0)),
            pl.BlockSpec((8, DIM), lambda i: (0, 0)),
            pl.BlockSpec((8, DIM), lambda i: (0, 0)),
            pl.BlockSpec((1, DIM), lambda i: (0, 0)),
            pl.BlockSpec((1, DIM), lambda i: (0, 0)),
        ],
        out_specs=pl.BlockSpec((ROW_BLK, DIM), lambda i: (i, 0)),
        out_shape=jax.ShapeDtypeStruct((N_NODES, DIM), jnp.float32),
    )(h3, ssum, ssq, g2, be2)


def kernel(h, edge_index, edge_weight, W, b, gamma, beta):
    src = edge_index[0].astype(jnp.int32)
    dst = edge_index[1].astype(jnp.int32)
    pad = E_PAD - N_EDGES
    src = jnp.concatenate([src, jnp.zeros((pad,), jnp.int32)])
    dst = jnp.concatenate([dst, jnp.full((pad,), N_ACC, jnp.int32)])
    ew = jnp.concatenate([edge_weight.astype(jnp.float32),
                          jnp.zeros((pad,), jnp.float32)])
    edata = jnp.stack(
        [src.reshape(NCHT, CH), dst.reshape(NCHT, CH),
         lax.bitcast_convert_type(ew, jnp.int32).reshape(NCHT, CH)],
        axis=1).reshape(-1)

    h2 = _project(h, W, b.reshape(1, DIM))
    agg = _sc_aggregate(h2, edata)
    h3, ssum, ssq = _combine(h2, agg[:N_NODES])
    return _bn_apply(h3, ssum, ssq, gamma.reshape(1, DIM),
                     beta.reshape(1, DIM))


# speedup vs baseline: 6.1316x; 6.1316x over previous
import functools

import jax
import jax.numpy as jnp
from jax import lax
from jax.experimental import pallas as pl
from jax.experimental.pallas import tpu as pltpu
from jax.experimental.pallas import tpu_sc as plsc

N_NODES = 10000
N_EDGES = 320000
DIM = 128
BN_EPS = 1e-5

NC = 2
NS = 16
NW = NC * NS
E_PAD = 327680
EPW = E_PAD // NW
CH = 128
NCH = EPW // CH
N_PAD = 10112
RPT = N_PAD // NS

ROW_BLK = 2000
N_BLKS = N_NODES // ROW_BLK


def _mm_body(h_ref, w_ref, b_ref, o_ref):
    o_ref[...] = (
        jnp.dot(h_ref[...], w_ref[...], preferred_element_type=jnp.float32)
        + b_ref[...]
    )


def _project(h, W, b2):
    return pl.pallas_call(
        _mm_body,
        grid=(N_BLKS,),
        in_specs=[
            pl.BlockSpec((ROW_BLK, DIM), lambda i: (i, 0)),
            pl.BlockSpec((DIM, DIM), lambda i: (0, 0)),
            pl.BlockSpec((1, DIM), lambda i: (0, 0)),
        ],
        out_specs=pl.BlockSpec((ROW_BLK, DIM), lambda i: (i, 0)),
        out_shape=jax.ShapeDtypeStruct((N_NODES, DIM), jnp.float32),
    )(h, W, b2)


def _sc_aggregate(h2, edata):
    mesh = plsc.VectorSubcoreMesh(
        core_axis_name="c", subcore_axis_name="s", num_cores=NC, num_subcores=NS
    )

    @functools.partial(
        pl.kernel,
        out_type=jax.ShapeDtypeStruct((NC, N_PAD, DIM), jnp.float32),
        mesh=mesh,
        scratch_types=[
            pltpu.VMEM((3, CH), jnp.int32),
            pltpu.VMEM((3, CH), jnp.int32),
            pltpu.VMEM((CH, DIM), jnp.float32),
            pltpu.VMEM((CH, DIM), jnp.float32),
            pltpu.VMEM((CH // 2,), jnp.int32),
            pltpu.VMEM((CH // 2,), jnp.int32),
            pltpu.VMEM((CH // 2,), jnp.int32),
            pltpu.VMEM((CH // 2,), jnp.int32),
            pltpu.VMEM((8, DIM), jnp.float32),
            pltpu.VMEM_SHARED((N_PAD, DIM), jnp.float32),
            pltpu.SemaphoreType.DMA,
            pltpu.SemaphoreType.DMA,
            pltpu.SemaphoreType.DMA,
            pltpu.SemaphoreType.DMA,
            pltpu.SemaphoreType.DMA,
            pltpu.SemaphoreType.DMA,
        ],
        compiler_params=pltpu.CompilerParams(needs_layout_passes=False),
    )
    def body(h2_hbm, ed_hbm, out_hbm,
             eb0, eb1, rows0, rows1, db0a, db0b, db1a, db1b, zbuf_v, agg_sh,
             se0, se1, sg0, sg1, ss0, ss1):
        c = lax.axis_index("c")
        s = lax.axis_index("s")
        wid = s * NC + c

        eb = (eb0, eb1)
        rows = (rows0, rows1)
        db = ((db0a, db0b), (db1a, db1b))
        se = (se0, se1)
        sg = (sg0, sg1)
        ss = (ss0, ss1)

        for r in range(8):
            for j in range(DIM // 16):
                zbuf_v[r, pl.ds(j * 16, 16)] = jnp.zeros((16,), jnp.float32)

        def zcopy(k, _):
            pltpu.sync_copy(zbuf_v, agg_sh.at[pl.ds(s * RPT + k * 8, 8)])
            return 0
        lax.fori_loop(0, RPT // 8, zcopy, 0)
        plsc.subcore_barrier()

        cbase = wid * NCH

        def fire_idx(k, p):
            pltpu.async_copy(ed_hbm.at[cbase + k], eb[p], se[p])

        def wait_idx(k, p):
            pltpu.make_async_copy(ed_hbm.at[cbase + k], eb[p], se[p]).wait()

        def fire_gather(p):
            pltpu.async_copy(h2_hbm.at[eb[p].at[0]], rows[p], sg[p])

        def wait_gather(p):
            pltpu.make_async_copy(h2_hbm.at[eb[p].at[0]], rows[p],
                                  sg[p]).wait()

        HF = CH // 2

        def fire_scatter(p, h):
            pltpu.async_copy(rows[p].at[pl.ds(h * HF, HF)],
                             agg_sh.at[db[p][h]], ss[p], add=True)

        def wait_scatter_both(p):
            for h in range(2):
                pltpu.make_async_copy(rows[p].at[pl.ds(h * HF, HF)],
                                      agg_sh.at[db[p][h]], ss[p]).wait()

        def compute_half(p, h):
            for j in range(HF // 16):
                db[p][h][pl.ds(j * 16, 16)] = (
                    eb[p][1, pl.ds(h * HF + j * 16, 16)])

            two = jnp.full((16,), 2, jnp.int32)

            def rowm2(i, _):
                wv = plsc.bitcast(
                    plsc.load_gather(eb[p], [two, lax.broadcast(i, (16,))]),
                    jnp.float32)
                for j in range(DIM // 16):
                    sl = pl.ds(j * 16, 16)
                    rows[p][i, sl] = rows[p][i, sl] * wv
                return 0
            lax.fori_loop(h * HF, (h + 1) * HF, rowm2, 0)

        fire_idx(0, 0)
        fire_idx(1, 1)
        wait_idx(0, 0)
        fire_gather(0)

        def pair(pr, _):
            for par in range(2):
                k = 2 * pr + par
                p = par
                q = 1 - par
                wait_gather(p)
                compute_half(p, 0)
                fire_scatter(p, 0)
                compute_half(p, 1)
                fire_scatter(p, 1)

                @pl.when(k + 2 < NCH)
                def _():
                    fire_idx(k + 2, p)

                @pl.when(k + 1 < NCH)
                def _():
                    wait_idx(k + 1, q)

                    @pl.when(k >= 1)
                    def _():
                        wait_scatter_both(q)
                    fire_gather(q)
            return 0
        lax.fori_loop(0, NCH // 2, pair, 0)

        wait_scatter_both(0)
        wait_scatter_both(1)
        plsc.subcore_barrier()
        pltpu.sync_copy(
            agg_sh.at[pl.ds(s * RPT, RPT)],
            out_hbm.at[c, pl.ds(s * RPT, RPT)],
        )

    return body(h2, edata)


def _comb_body(h2_ref, p0_ref, p1_ref, h3_ref, sum_ref, sq_ref):
    i = pl.program_id(0)
    x = h2_ref[...] + p0_ref[...] + p1_ref[...]
    h3_ref[...] = x

    @pl.when(i == 0)
    def _():
        sum_ref[...] = jnp.zeros_like(sum_ref)
        sq_ref[...] = jnp.zeros_like(sq_ref)

    sum_ref[0:1, :] += jnp.sum(x, axis=0, keepdims=True)
    sq_ref[0:1, :] += jnp.sum(x * x, axis=0, keepdims=True)


def _combine(h2, p0, p1):
    return pl.pallas_call(
        _comb_body,
        grid=(N_BLKS,),
        in_specs=[
            pl.BlockSpec((ROW_BLK, DIM), lambda i: (i, 0)),
            pl.BlockSpec((ROW_BLK, DIM), lambda i: (i, 0)),
            pl.BlockSpec((ROW_BLK, DIM), lambda i: (i, 0)),
        ],
        out_specs=[
            pl.BlockSpec((ROW_BLK, DIM), lambda i: (i, 0)),
            pl.BlockSpec((8, DIM), lambda i: (0, 0)),
            pl.BlockSpec((8, DIM), lambda i: (0, 0)),
        ],
        out_shape=[
            jax.ShapeDtypeStruct((N_NODES, DIM), jnp.float32),
            jax.ShapeDtypeStruct((8, DIM), jnp.float32),
            jax.ShapeDtypeStruct((8, DIM), jnp.float32),
        ],
    )(h2, p0, p1)


def _bn_body(h3_ref, sum_ref, sq_ref, g_ref, be_ref, o_ref):
    n = jnp.float32(N_NODES)
    mean = sum_ref[0:1, :] / n
    var = sq_ref[0:1, :] / n - mean * mean
    inv = lax.rsqrt(var + BN_EPS)
    o_ref[...] = g_ref[...] * (h3_ref[...] - mean) * inv + be_ref[...]


def _bn_apply(h3, ssum, ssq, g2, be2):
    return pl.pallas_call(
        _bn_body,
        grid=(N_BLKS,),
        in_specs=[
            pl.BlockSpec((ROW_BLK, DIM), lambda i: (i, 0)),
            pl.BlockSpec((8, DIM), lambda i: (0, 0)),
            pl.BlockSpec((8, DIM), lambda i: (0, 0)),
            pl.BlockSpec((1, DIM), lambda i: (0, 0)),
            pl.BlockSpec((1, DIM), lambda i: (0, 0)),
        ],
        out_specs=pl.BlockSpec((ROW_BLK, DIM), lambda i: (i, 0)),
        out_shape=jax.ShapeDtypeStruct((N_NODES, DIM), jnp.float32),
    )(h3, ssum, ssq, g2, be2)


def kernel(h, edge_index, edge_weight, W, b, gamma, beta):
    src = edge_index[0].astype(jnp.int32)
    dst = edge_index[1].astype(jnp.int32)
    pad = E_PAD - N_EDGES
    src = jnp.concatenate([src, jnp.zeros((pad,), jnp.int32)])
    dst = jnp.concatenate([dst, jnp.zeros((pad,), jnp.int32)])
    ew = jnp.concatenate([edge_weight.astype(jnp.float32),
                          jnp.zeros((pad,), jnp.float32)])
    nch_tot = E_PAD // CH
    edata = jnp.stack(
        [src.reshape(nch_tot, CH), dst.reshape(nch_tot, CH),
         lax.bitcast_convert_type(ew, jnp.int32).reshape(nch_tot, CH)],
        axis=1)

    h2 = _project(h, W, b.reshape(1, DIM))
    parts = _sc_aggregate(h2, edata)
    h3, ssum, ssq = _combine(h2, parts[0, :N_NODES], parts[1, :N_NODES])
    return _bn_apply(h3, ssum, ssq, gamma.reshape(1, DIM),
                     beta.reshape(1, DIM))
